# baseline (speedup 1.0000x reference)
import functools

import jax
import jax.numpy as jnp
import numpy as np
from jax import lax
from jax.experimental import pallas as pl
from jax.experimental.pallas import tpu as pltpu
from jax.experimental.pallas import tpu_sc as plsc

_HIDDEN = 128
_TABLE_ROWS = 1000
_IDX_CHUNK = 128
_LANES = 16


def _sc_gather(table, idx3, n_rows):
    nw, n_ch, ch = idx3.shape
    rows_per_w = n_ch * ch
    mesh = plsc.VectorSubcoreMesh(core_axis_name="c", subcore_axis_name="s")

    @functools.partial(
        pl.kernel,
        mesh=mesh,
        out_type=jax.ShapeDtypeStruct((n_rows, _HIDDEN), jnp.float32),
        scratch_types=[
            pltpu.VMEM((n_ch, _IDX_CHUNK), jnp.int32),
            pltpu.VMEM((rows_per_w, _HIDDEN), jnp.float32),
            pltpu.VMEM((128, _HIDDEN), jnp.float32),
            pltpu.VMEM_SHARED((_TABLE_ROWS, _HIDDEN), jnp.float32),
            pltpu.SemaphoreType.DMA,
            pltpu.SemaphoreType.DMA,
        ],
    )
    def gather_kernel(table_hbm, idx_hbm, out_hbm, idx_v, rows_v, stage_v, tab_sh,
                      gsem, ssem):
        num_cores = lax.axis_size("c")
        sid = lax.axis_index("s")
        wid = sid * num_cores + lax.axis_index("c")
        base = wid * rows_per_w
        @pl.when(sid < 8)
        def _stage():
            start = jnp.where(sid < 7, sid * 128, _TABLE_ROWS - 128)
            pltpu.sync_copy(table_hbm.at[pl.ds(start, 128)], stage_v)
            pltpu.sync_copy(stage_v, tab_sh.at[pl.ds(start, 128)])

        pltpu.sync_copy(idx_hbm.at[wid], idx_v)
        plsc.subcore_barrier()
        gathers = [
            pltpu.async_copy(
                tab_sh.at[idx_v.at[c]],
                rows_v.at[pl.ds(c * _IDX_CHUNK, _IDX_CHUNK)],
                gsem,
            )
            for c in range(n_ch)
        ]
        scatters = []
        for c in range(n_ch):
            gathers[c].wait()
            scatters.append(
                pltpu.async_copy(
                    rows_v.at[pl.ds(c * _IDX_CHUNK, _IDX_CHUNK)],
                    out_hbm.at[pl.ds(base + c * _IDX_CHUNK, _IDX_CHUNK)],
                    ssem,
                )
            )
        for s in scatters:
            s.wait()

    return gather_kernel(table, idx3)


def _add_body(f_ref, p_ref, o_ref):
    pos = p_ref[...]
    o_ref[...] = f_ref[...] + pos[None, :, :]


def _add_body_aliased(_prev_ref, f_ref, p_ref, o_ref):
    pos = p_ref[...]
    o_ref[...] = f_ref[...] + pos[None, :, :]


def _tc_add_half(ft, pos_half, out_prev, half):
    t, n, d = ft.shape
    hn = pos_half.shape[0]
    grid = (1, t)
    if out_prev is None:
        return pl.pallas_call(
            _add_body,
            grid=grid,
            in_specs=[
                pl.BlockSpec((1, hn, d), lambda j, i, h=half: (i, h, 0)),
                pl.BlockSpec((hn, d), lambda j, i: (0, 0)),
            ],
            out_specs=pl.BlockSpec((1, hn, d), lambda j, i, h=half: (i, h, 0)),
            out_shape=jax.ShapeDtypeStruct((t, n, d), ft.dtype),
        )(ft, pos_half)
    return pl.pallas_call(
        _add_body_aliased,
        grid=grid,
        in_specs=[
            pl.BlockSpec(memory_space=pl.ANY),
            pl.BlockSpec((1, hn, d), lambda j, i, h=half: (i, h, 0)),
            pl.BlockSpec((hn, d), lambda j, i: (0, 0)),
        ],
        out_specs=pl.BlockSpec((1, hn, d), lambda j, i, h=half: (i, h, 0)),
        out_shape=jax.ShapeDtypeStruct((t, n, d), ft.dtype),
        input_output_aliases={0: 0},
    )(out_prev, ft, pos_half)


def kernel(features, temporal_embedding):
    n = features.shape[0]
    temporal_pos = jnp.linspace(0.0, 1.0, n, dtype=features.dtype)
    idx = jnp.clip(temporal_pos * _TABLE_ROWS, 0, _TABLE_ROWS - 1).astype(jnp.int32)

    info = plsc.get_sparse_core_info()
    nw = info.num_cores * info.num_subcores
    half_n = n // 2
    idx_a = lax.slice(idx, (0,), (half_n,)).reshape(nw, -1, _IDX_CHUNK)
    idx_b = lax.slice(idx, (half_n,), (n,)).reshape(nw, -1, _IDX_CHUNK)

    pos_a = _sc_gather(temporal_embedding, idx_a, half_n)
    pos_b = _sc_gather(temporal_embedding, idx_b, half_n)

    ft = jnp.transpose(features, (1, 0, 2))
    out_t = _tc_add_half(ft, pos_a, None, half=0)
    out_t = _tc_add_half(ft, pos_b, out_t, half=1)
    return jnp.transpose(out_t, (1, 0, 2))
